# baseline (speedup 1.0000x reference)
import functools

import jax
import jax.numpy as jnp
from jax import lax
from jax.experimental import pallas as pl
from jax.experimental.pallas import tpu as pltpu
from jax.experimental.pallas import tpu_sc as plsc

_N, _D, _E, _G = 10000, 128, 320000, 64
_NP = 10240
_NC, _NS = 2, 16
_NW = _NC * _NS
_K = 80
_CH = 128
_EPT = _K * _CH
_EPAD = _EPT * _NW
_RPT = _NP // _NS
_BN = 1024


def _sc_agg(h, edges):
    mesh = plsc.VectorSubcoreMesh(core_axis_name="c", subcore_axis_name="s")

    @functools.partial(
        pl.kernel,
        out_type=jax.ShapeDtypeStruct((_NC * _NP, _D), jnp.float32),
        mesh=mesh,
        scratch_types=[
            [pltpu.VMEM((2, _K), jnp.int32) for _ in range(4)],
            [pltpu.VMEM((_K, _D), jnp.float32) for _ in range(4)],
            pltpu.VMEM_SHARED((_NP, _D), jnp.float32),
            [pltpu.SemaphoreType.DMA for _ in range(4)],
            [pltpu.SemaphoreType.DMA for _ in range(4)],
            [pltpu.SemaphoreType.DMA for _ in range(4)],
        ],
    )
    def k(h_hbm, e_hbm, out_hbm, ib, rv, acc_sh, sx, sg, ss):
        c = lax.axis_index("c")
        s = lax.axis_index("s")
        wid = c * _NS + s
        pltpu.sync_copy(h_hbm.at[pl.ds(s * _RPT, _RPT)],
                        acc_sh.at[pl.ds(s * _RPT, _RPT)])
        plsc.subcore_barrier()

        def wait_idx(k):
            pltpu.make_async_copy(e_hbm.at[0, 0], ib[k], sx[k]).wait()

        def wait_rows(k):
            pltpu.make_async_copy(h_hbm.at[pl.ds(0, _K)], rv[k], sg[k]).wait()

        def wait_scat(k):
            pltpu.make_async_copy(rv[k], acc_sh.at[pl.ds(0, _K)], ss[k]).wait()

        def start_idx(ci, k):
            pltpu.async_copy(e_hbm.at[ci, wid], ib[k], sx[k])

        def start_gather(k):
            pltpu.async_copy(h_hbm.at[ib[k].at[0]], rv[k], sg[k])

        def start_scat(k):
            pltpu.async_copy(rv[k], acc_sh.at[ib[k].at[1]], ss[k], add=True)

        start_idx(0, 0)
        start_idx(1, 1)
        wait_idx(0)
        start_gather(0)
        wait_idx(1)
        start_gather(1)

        def body(j, carry):
            b = 4 * j
            for m in range(4):
                k = m
                k2 = (m + 2) % 4
                wait_rows(k)
                start_scat(k)
                if m < 2:
                    @pl.when(j > 0)
                    def _():
                        wait_scat(k2)
                    start_idx(b + m + 2, k2)
                    wait_idx(k2)
                    start_gather(k2)
                else:
                    wait_scat(k2)

                    @pl.when(j < _CH // 4 - 1)
                    def _():
                        start_idx(b + m + 2, k2)
                        wait_idx(k2)
                        start_gather(k2)
            return carry

        lax.fori_loop(0, _CH // 4, body, 0)
        wait_scat(2)
        wait_scat(3)
        plsc.subcore_barrier()
        pltpu.sync_copy(acc_sh.at[pl.ds(s * _RPT, _RPT)],
                        out_hbm.at[pl.ds(c * _NP + s * _RPT, _RPT)])

    return k(h, edges)


def _mlp_body(scale_ref, h_ref, p0_ref, p1_ref, w1_ref, b1_ref, w2_ref, b2_ref):
    z = h_ref[...] * scale_ref[0, 0] + p0_ref[...] + p1_ref[...]
    y = jnp.dot(z, w1_ref[...], preferred_element_type=jnp.float32) + b1_ref[...]
    y = jnp.maximum(y, 0.0)
    return jnp.dot(y, w2_ref[...], preferred_element_type=jnp.float32) + b2_ref[...]


_ROW_SPECS = [
    pl.BlockSpec(memory_space=pltpu.SMEM),
    pl.BlockSpec((_BN, _D), lambda i: (i, 0)),
    pl.BlockSpec((_BN, _D), lambda i: (i, 0)),
    pl.BlockSpec((_BN, _D), lambda i: (i + _NP // _BN, 0)),
    pl.BlockSpec((_D, _D), lambda i: (0, 0)),
    pl.BlockSpec((1, _D), lambda i: (0, 0)),
    pl.BlockSpec((_D, _D), lambda i: (0, 0)),
    pl.BlockSpec((1, _D), lambda i: (0, 0)),
]


def _tc_mlp(h, p0, p1, scale, w1, b1, w2, b2):
    def body(scale_ref, h_ref, p0_ref, p1_ref, w1_ref, b1_ref, w2_ref, b2_ref,
             o_ref):
        o = _mlp_body(scale_ref, h_ref, p0_ref, p1_ref, w1_ref, b1_ref,
                      w2_ref, b2_ref)
        o_ref[...] = jnp.maximum(o, 0.0)

    return pl.pallas_call(
        body,
        grid=(_NP // _BN,),
        in_specs=_ROW_SPECS,
        out_specs=pl.BlockSpec((_BN, _D), lambda i: (i, 0)),
        out_shape=jax.ShapeDtypeStruct((_NP, _D), jnp.float32),
    )(scale, h, p0, p1, w1, b1, w2, b2)


def _tc_mlp_pool(h, p0, p1, scale, w1, b1, w2, b2, batm):
    def body(scale_ref, h_ref, p0_ref, p1_ref, w1_ref, b1_ref, w2_ref, b2_ref,
             bat_ref, o_ref):
        i = pl.program_id(0)
        o = _mlp_body(scale_ref, h_ref, p0_ref, p1_ref, w1_ref, b1_ref,
                      w2_ref, b2_ref)
        seg = bat_ref[0]
        gids = lax.broadcasted_iota(jnp.int32, (_G, _BN), 0)
        onehot = (gids == seg).astype(jnp.float32)
        part = jnp.dot(onehot, o, preferred_element_type=jnp.float32)

        @pl.when(i == 0)
        def _():
            o_ref[...] = jnp.zeros_like(o_ref)

        o_ref[...] += part

    return pl.pallas_call(
        body,
        grid=(_NP // _BN,),
        in_specs=_ROW_SPECS + [pl.BlockSpec((1, 1, _BN), lambda i: (i, 0, 0))],
        out_specs=pl.BlockSpec((_G, _D), lambda i: (0, 0)),
        out_shape=jax.ShapeDtypeStruct((_G, _D), jnp.float32),
    )(scale, h, p0, p1, w1, b1, w2, b2, batm)


def kernel(x, edge_index, batch, eps0, W1_0, b1_0, W2_0, b2_0,
           eps1, W1_1, b1_1, W2_1, b2_1, eps2, W1_2, b1_2, W2_2, b2_2):
    pad = _EPAD - _E
    srcp = jnp.concatenate(
        [edge_index[0], jnp.zeros((pad,), jnp.int32)]).reshape(_CH, _NW, _K)
    dump = _N + jnp.arange(pad, dtype=jnp.int32) % (_NP - _N)
    dstp = jnp.concatenate([edge_index[1], dump]).reshape(_CH, _NW, _K)
    edges = jnp.stack([srcp, dstp], axis=2)
    batm = jnp.concatenate(
        [batch, jnp.full((_NP - _N,), _G, jnp.int32)]).reshape(_NP // _BN, 1, _BN)

    h = jnp.pad(x, ((0, _NP - _N), (0, 0)))
    layers = [(eps0, W1_0, b1_0, W2_0, b2_0),
              (eps1, W1_1, b1_1, W2_1, b2_1),
              (eps2, W1_2, b1_2, W2_2, b2_2)]
    for l, (eps, w1, b1, w2, b2) in enumerate(layers):
        parts = _sc_agg(h, edges)
        scale = (eps - 1.0).reshape(1, 1)
        b1r, b2r = b1.reshape(1, _D), b2.reshape(1, _D)
        if l < 2:
            h = _tc_mlp(h, parts, parts, scale, w1, b1r, w2, b2r)
        else:
            return _tc_mlp_pool(h, parts, parts, scale, w1, b1r, w2, b2r, batm)


# speedup vs baseline: 1.9111x; 1.9111x over previous
import functools

import jax
import jax.numpy as jnp
from jax import lax
from jax.experimental import pallas as pl
from jax.experimental.pallas import tpu as pltpu
from jax.experimental.pallas import tpu_sc as plsc

_N, _D, _E, _G = 10000, 128, 320000, 64
_NP = 10240
_NC, _NS = 2, 16
_NW = _NC * _NS
_K = 120
_CH = 84
_EPT = _K * _CH
_EPAD = _EPT * _NW
_RPT = _NP // _NS
_BN = 1024


def _sc_agg(h, edges):
    mesh = plsc.VectorSubcoreMesh(core_axis_name="c", subcore_axis_name="s")

    @functools.partial(
        pl.kernel,
        out_type=jax.ShapeDtypeStruct((_NC * _NP, _D), jnp.float32),
        mesh=mesh,
        scratch_types=[
            [pltpu.VMEM((2, _K), jnp.int32) for _ in range(3)],
            [pltpu.VMEM((_K, _D), jnp.float32) for _ in range(3)],
            pltpu.VMEM_SHARED((_NP, _D), jnp.float32),
            [pltpu.SemaphoreType.DMA for _ in range(3)],
            [pltpu.SemaphoreType.DMA for _ in range(3)],
            [pltpu.SemaphoreType.DMA for _ in range(3)],
        ],
    )
    def k(h_hbm, e_hbm, out_hbm, ib, rv, acc_sh, sx, sg, ss):
        c = lax.axis_index("c")
        s = lax.axis_index("s")
        wid = c * _NS + s
        pltpu.sync_copy(h_hbm.at[pl.ds(s * _RPT, _RPT)],
                        acc_sh.at[pl.ds(s * _RPT, _RPT)])
        plsc.subcore_barrier()

        def wait_idx(k):
            pltpu.make_async_copy(e_hbm.at[0, 0], ib[k], sx[k]).wait()

        def wait_rows(k):
            pltpu.make_async_copy(h_hbm.at[pl.ds(0, _K)], rv[k], sg[k]).wait()

        def wait_scat(k):
            pltpu.make_async_copy(rv[k], acc_sh.at[pl.ds(0, _K)], ss[k]).wait()

        def start_idx(ci, k):
            pltpu.async_copy(e_hbm.at[ci, wid], ib[k], sx[k])

        def start_gather(k):
            pltpu.async_copy(h_hbm.at[ib[k].at[0]], rv[k], sg[k])

        def start_scat(k):
            pltpu.async_copy(rv[k], acc_sh.at[ib[k].at[1]], ss[k], add=True)

        start_idx(0, 0)
        start_idx(1, 1)
        wait_idx(0)
        start_gather(0)
        wait_idx(1)
        start_gather(1)

        def body(j, carry):
            b = 3 * j
            for m in range(3):
                k = m
                k2 = (m + 2) % 3
                wait_rows(k)
                start_scat(k)
                if m < 1:
                    @pl.when(j > 0)
                    def _():
                        wait_scat(k2)
                    start_idx(b + m + 2, k2)
                    wait_idx(k2)
                    start_gather(k2)
                else:
                    wait_scat(k2)

                    @pl.when(j < _CH // 3 - 1)
                    def _():
                        start_idx(b + m + 2, k2)
                        wait_idx(k2)
                        start_gather(k2)
            return carry

        lax.fori_loop(0, _CH // 3, body, 0)
        wait_scat(2)
        plsc.subcore_barrier()
        pltpu.sync_copy(acc_sh.at[pl.ds(s * _RPT, _RPT)],
                        out_hbm.at[pl.ds(c * _NP + s * _RPT, _RPT)])

    return k(h, edges)


def _mlp_body(scale_ref, h_ref, p0_ref, p1_ref, w1_ref, b1_ref, w2_ref, b2_ref):
    z = h_ref[...] * scale_ref[0, 0] + p0_ref[...] + p1_ref[...]
    y = jnp.dot(z, w1_ref[...], preferred_element_type=jnp.float32) + b1_ref[...]
    y = jnp.maximum(y, 0.0)
    return jnp.dot(y, w2_ref[...], preferred_element_type=jnp.float32) + b2_ref[...]


_ROW_SPECS = [
    pl.BlockSpec(memory_space=pltpu.SMEM),
    pl.BlockSpec((_BN, _D), lambda i: (i, 0)),
    pl.BlockSpec((_BN, _D), lambda i: (i, 0)),
    pl.BlockSpec((_BN, _D), lambda i: (i + _NP // _BN, 0)),
    pl.BlockSpec((_D, _D), lambda i: (0, 0)),
    pl.BlockSpec((1, _D), lambda i: (0, 0)),
    pl.BlockSpec((_D, _D), lambda i: (0, 0)),
    pl.BlockSpec((1, _D), lambda i: (0, 0)),
]


def _tc_mlp(h, p0, p1, scale, w1, b1, w2, b2):
    def body(scale_ref, h_ref, p0_ref, p1_ref, w1_ref, b1_ref, w2_ref, b2_ref,
             o_ref):
        o = _mlp_body(scale_ref, h_ref, p0_ref, p1_ref, w1_ref, b1_ref,
                      w2_ref, b2_ref)
        o_ref[...] = jnp.maximum(o, 0.0)

    return pl.pallas_call(
        body,
        grid=(_NP // _BN,),
        in_specs=_ROW_SPECS,
        out_specs=pl.BlockSpec((_BN, _D), lambda i: (i, 0)),
        out_shape=jax.ShapeDtypeStruct((_NP, _D), jnp.float32),
    )(scale, h, p0, p1, w1, b1, w2, b2)


def _tc_mlp_pool(h, p0, p1, scale, w1, b1, w2, b2, batm):
    def body(scale_ref, h_ref, p0_ref, p1_ref, w1_ref, b1_ref, w2_ref, b2_ref,
             bat_ref, o_ref):
        i = pl.program_id(0)
        o = _mlp_body(scale_ref, h_ref, p0_ref, p1_ref, w1_ref, b1_ref,
                      w2_ref, b2_ref)
        seg = bat_ref[0]
        gids = lax.broadcasted_iota(jnp.int32, (_G, _BN), 0)
        onehot = (gids == seg).astype(jnp.float32)
        part = jnp.dot(onehot, o, preferred_element_type=jnp.float32)

        @pl.when(i == 0)
        def _():
            o_ref[...] = jnp.zeros_like(o_ref)

        o_ref[...] += part

    return pl.pallas_call(
        body,
        grid=(_NP // _BN,),
        in_specs=_ROW_SPECS + [pl.BlockSpec((1, 1, _BN), lambda i: (i, 0, 0))],
        out_specs=pl.BlockSpec((_G, _D), lambda i: (0, 0)),
        out_shape=jax.ShapeDtypeStruct((_G, _D), jnp.float32),
    )(scale, h, p0, p1, w1, b1, w2, b2, batm)


def kernel(x, edge_index, batch, eps0, W1_0, b1_0, W2_0, b2_0,
           eps1, W1_1, b1_1, W2_1, b2_1, eps2, W1_2, b1_2, W2_2, b2_2):
    pad = _EPAD - _E
    srcp = jnp.concatenate(
        [edge_index[0], jnp.zeros((pad,), jnp.int32)]).reshape(_CH, _NW, _K)
    dump = _N + jnp.arange(pad, dtype=jnp.int32) % (_NP - _N)
    dstp = jnp.concatenate([edge_index[1], dump]).reshape(_CH, _NW, _K)
    edges = jnp.stack([srcp, dstp], axis=2)
    batm = jnp.concatenate(
        [batch, jnp.full((_NP - _N,), _G, jnp.int32)]).reshape(_NP // _BN, 1, _BN)

    h = jnp.pad(x, ((0, _NP - _N), (0, 0)))
    layers = [(eps0, W1_0, b1_0, W2_0, b2_0),
              (eps1, W1_1, b1_1, W2_1, b2_1),
              (eps2, W1_2, b1_2, W2_2, b2_2)]
    for l, (eps, w1, b1, w2, b2) in enumerate(layers):
        parts = _sc_agg(h, edges)
        scale = (eps - 1.0).reshape(1, 1)
        b1r, b2r = b1.reshape(1, _D), b2.reshape(1, _D)
        if l < 2:
            h = _tc_mlp(h, parts, parts, scale, w1, b1r, w2, b2r)
        else:
            return _tc_mlp_pool(h, parts, parts, scale, w1, b1r, w2, b2r, batm)
